# Initial kernel scaffold; baseline (speedup 1.0000x reference)
import jax
import jax.numpy as jnp
from jax.experimental import pallas as pl


def kernel(node_type, edge_index, edge_type, W1, b1, W2, b2):
    raise NotImplementedError("write your pallas kernel here")


# speedup vs baseline: 170.5692x; 170.5692x over previous
import functools

import jax
import jax.numpy as jnp
from jax import lax
from jax.experimental import pallas as pl
from jax.experimental.pallas import tpu as pltpu
from jax.experimental.pallas import tpu_sc as plsc

_N_NODES = 100000
_N_EDGES = 6400000
_NUM_ET = 38
_HID = 128
_TBL = 640
_LANES = 128
_ROWS = _N_EDGES // _LANES
_CHUNK = 8
_NCHUNK = _ROWS // _CHUNK
_NW = 32
_ZC = 2000
_NZC = _N_NODES // _ZC


def _table_body(w1_ref, b1_ref, w2_ref, b2_ref, o_ref):
    row = lax.broadcasted_iota(jnp.int32, (_TBL, _HID), 0)
    col = lax.broadcasted_iota(jnp.int32, (_TBL, _HID), 1)
    et = row // 16
    ht = (row // 4) % 4
    tt = row % 4
    feat = ((col == et)
            | ((col >= _NUM_ET) & (col < _NUM_ET + 4) & (col - _NUM_ET == ht))
            | ((col >= _NUM_ET + 4) & (col < _NUM_ET + 8)
               & (col - (_NUM_ET + 4) == tt))).astype(jnp.float32)
    h = jnp.dot(feat, w1_ref[...], preferred_element_type=jnp.float32,
                precision="highest") + b1_ref[...]
    g = jax.nn.gelu(h)
    o = jnp.dot(g, w2_ref[...], preferred_element_type=jnp.float32,
                precision="highest") + b2_ref[...]
    o_ref[...] = jax.nn.sigmoid(o)


def _make_table(W1, b1, W2, b2):
    w1p = jnp.zeros((_HID, _HID), jnp.float32).at[: W1.shape[0]].set(W1)
    w2p = jnp.zeros((_HID, _HID), jnp.float32).at[:, :1].set(W2)
    b1r = b1.reshape(1, _HID)
    b2r = jnp.broadcast_to(b2.reshape(1, 1), (1, _HID))
    out = pl.pallas_call(
        _table_body,
        out_shape=jax.ShapeDtypeStruct((_TBL, _HID), jnp.float32),
    )(w1p, b1r, w2p, b2r)
    return out[:, 0]


def _worker_bounds(wid):
    base = _NCHUNK // _NW
    extra = _NCHUNK - base * _NW
    start = wid * base + jnp.minimum(wid, extra)
    cnt = base + jnp.where(wid < extra, 1, 0)
    return start, cnt


def _fill_zeros(zbuf):
    zeros16 = jnp.zeros((16,), jnp.float32)
    for i in range(_ZC // 16):
        zbuf[pl.ds(i * 16, 16)] = zeros16


def _zero_acc(acc, zbuf, sid):
    for j in range(4):
        k = sid + j * 16
        @pl.when(k < _NZC)
        def _():
            pltpu.sync_copy(zbuf, acc.at[pl.ds(k * _ZC, _ZC)])


def _write_partial(acc, zbuf, out_hbm, cid, sid):
    for j in range(4):
        k = sid + j * 16
        @pl.when(k < _NZC)
        def _():
            pltpu.sync_copy(acc.at[pl.ds(k * _ZC, _ZC)], zbuf)
            pltpu.sync_copy(zbuf, out_hbm.at[pl.ds(cid * _N_NODES + k * _ZC, _ZC)])


def _edge_pass(src_hbm, dst_hbm, srcb, dstb, valsb, acc, wid, gather_vals):
    start, cnt = _worker_bounds(wid)

    def chunk_body(ci, _):
        r0 = (start + ci) * _CHUNK
        pltpu.sync_copy(src_hbm.at[pl.ds(r0, _CHUNK)], srcb)
        pltpu.sync_copy(dst_hbm.at[pl.ds(r0, _CHUNK)], dstb)
        gather_vals(r0)
        for j in range(_CHUNK):
            pltpu.sync_copy(valsb.at[j], acc.at[dstb.at[j]], add=True)
        return 0

    lax.fori_loop(0, cnt, chunk_body, 0)


def _hop1_kernel(nt_hbm, src_hbm, dst_hbm, et_hbm, tbl_hbm, out_hbm,
                 nt_v, tbl_v, srcb, dstb, etb, valsb, zbuf, acc):
    cid = lax.axis_index("c")
    sid = lax.axis_index("s")
    wid = sid * 2 + cid

    pltpu.sync_copy(nt_hbm, nt_v)
    pltpu.sync_copy(tbl_hbm, tbl_v)
    _fill_zeros(zbuf)
    _zero_acc(acc, zbuf, sid)
    plsc.subcore_barrier()

    def gather_vals(r0):
        pltpu.sync_copy(et_hbm.at[pl.ds(r0, _CHUNK)], etb)
        for j in range(_CHUNK):
            for b in range(_LANES // 16):
                s16 = srcb[j, pl.ds(b * 16, 16)]
                d16 = dstb[j, pl.ds(b * 16, 16)]
                e16 = etb[j, pl.ds(b * 16, 16)]
                ht = plsc.load_gather(nt_v, [s16])
                tt = plsc.load_gather(nt_v, [d16])
                combo = e16 * 16 + ht * 4 + tt
                valsb[j, pl.ds(b * 16, 16)] = plsc.load_gather(tbl_v, [combo])

    _edge_pass(src_hbm, dst_hbm, srcb, dstb, valsb, acc, wid, gather_vals)
    plsc.subcore_barrier()
    _write_partial(acc, zbuf, out_hbm, cid, sid)


def _hop_kernel(nparts, emit_x, *refs):
    if nparts == 3:
        (src_hbm, dst_hbm, parts_hbm, extra_hbm, *rest) = refs
    else:
        (src_hbm, dst_hbm, parts_hbm, *rest) = refs
        extra_hbm = None
    if emit_x:
        (xout_hbm, out_hbm, x_v, srcb, dstb, valsb, t0, t1, zbuf, acc) = rest
    else:
        (out_hbm, x_v, srcb, dstb, valsb, t0, t1, zbuf, acc) = rest
    cid = lax.axis_index("c")
    sid = lax.axis_index("s")
    wid = sid * 2 + cid

    def stage_body(k, _):
        pltpu.sync_copy(parts_hbm.at[pl.ds(k * _ZC, _ZC)], t0)
        pltpu.sync_copy(parts_hbm.at[pl.ds(_N_NODES + k * _ZC, _ZC)], t1)
        if extra_hbm is not None:
            pltpu.sync_copy(extra_hbm.at[pl.ds(k * _ZC, _ZC)], zbuf)
        for i in range(_ZC // 16):
            v = t0[pl.ds(i * 16, 16)] + t1[pl.ds(i * 16, 16)]
            if extra_hbm is not None:
                v = v + zbuf[pl.ds(i * 16, 16)]
            x_v[pl.ds(k * _ZC + i * 16, 16)] = v
        if emit_x:
            @pl.when(lax.rem(k, _NW) == wid)
            def _():
                pltpu.sync_copy(x_v.at[pl.ds(k * _ZC, _ZC)],
                                xout_hbm.at[pl.ds(k * _ZC, _ZC)])
        return 0

    lax.fori_loop(0, _NZC, stage_body, 0)

    _fill_zeros(zbuf)
    _zero_acc(acc, zbuf, sid)
    plsc.subcore_barrier()

    def gather_vals(r0):
        for j in range(_CHUNK):
            for b in range(_LANES // 16):
                s16 = srcb[j, pl.ds(b * 16, 16)]
                valsb[j, pl.ds(b * 16, 16)] = plsc.load_gather(x_v, [s16])

    _edge_pass(src_hbm, dst_hbm, srcb, dstb, valsb, acc, wid, gather_vals)
    plsc.subcore_barrier()
    _write_partial(acc, zbuf, out_hbm, cid, sid)


def _final_kernel(p_hbm, x1_hbm, out_hbm, t0, t1, t2, sbuf):
    cid = lax.axis_index("c")
    sid = lax.axis_index("s")
    wid = sid * 2 + cid
    for j in range(2):
        k = wid + j * _NW
        @pl.when(k < _NZC)
        def _():
            pltpu.sync_copy(p_hbm.at[pl.ds(k * _ZC, _ZC)], t0)
            pltpu.sync_copy(p_hbm.at[pl.ds(_N_NODES + k * _ZC, _ZC)], t1)
            pltpu.sync_copy(x1_hbm.at[pl.ds(k * _ZC, _ZC)], t2)
            for i in range(_ZC // 16):
                sbuf[pl.ds(i * 16, 16)] = (t0[pl.ds(i * 16, 16)]
                                           + t1[pl.ds(i * 16, 16)]
                                           + t2[pl.ds(i * 16, 16)])
            pltpu.sync_copy(sbuf, out_hbm.at[pl.ds(k * _ZC, _ZC)])


def _sc_mesh():
    return plsc.VectorSubcoreMesh(core_axis_name="c", subcore_axis_name="s")


_SC_PARAMS = pltpu.CompilerParams(needs_layout_passes=False)


@jax.jit
def kernel(node_type, edge_index, edge_type, W1, b1, W2, b2):
    table = _make_table(W1, b1, W2, b2)

    src2d = edge_index[0].reshape(_ROWS, _LANES)
    dst2d = edge_index[1].reshape(_ROWS, _LANES)
    et2d = edge_type.reshape(_ROWS, _LANES)

    f32 = jnp.float32
    i32 = jnp.int32

    hop1 = pl.kernel(
        _hop1_kernel,
        out_type=jax.ShapeDtypeStruct((2 * _N_NODES,), f32),
        mesh=_sc_mesh(),
        compiler_params=_SC_PARAMS,
        scratch_types=[
            pltpu.VMEM((_N_NODES,), i32),
            pltpu.VMEM((_TBL,), f32),
            pltpu.VMEM((_CHUNK, _LANES), i32),
            pltpu.VMEM((_CHUNK, _LANES), i32),
            pltpu.VMEM((_CHUNK, _LANES), i32),
            pltpu.VMEM((_CHUNK, _LANES), f32),
            pltpu.VMEM((_ZC,), f32),
            pltpu.VMEM_SHARED((_N_NODES,), f32),
        ],
    )
    q = hop1(node_type, src2d, dst2d, et2d, table)

    hop_scratch = [
        pltpu.VMEM((_N_NODES,), f32),
        pltpu.VMEM((_CHUNK, _LANES), i32),
        pltpu.VMEM((_CHUNK, _LANES), i32),
        pltpu.VMEM((_CHUNK, _LANES), f32),
        pltpu.VMEM((_ZC,), f32),
        pltpu.VMEM((_ZC,), f32),
        pltpu.VMEM((_ZC,), f32),
        pltpu.VMEM_SHARED((_N_NODES,), f32),
    ]

    hop2 = pl.kernel(
        functools.partial(_hop_kernel, 2, True),
        out_type=(jax.ShapeDtypeStruct((_N_NODES,), f32),
                  jax.ShapeDtypeStruct((2 * _N_NODES,), f32)),
        mesh=_sc_mesh(),
        compiler_params=_SC_PARAMS,
        scratch_types=hop_scratch,
    )
    x1_full, p2 = hop2(src2d, dst2d, q)

    hop3 = pl.kernel(
        functools.partial(_hop_kernel, 3, False),
        out_type=jax.ShapeDtypeStruct((2 * _N_NODES,), f32),
        mesh=_sc_mesh(),
        compiler_params=_SC_PARAMS,
        scratch_types=list(hop_scratch),
    )
    p3 = hop3(src2d, dst2d, p2, x1_full)

    final = pl.kernel(
        _final_kernel,
        out_type=jax.ShapeDtypeStruct((_N_NODES,), f32),
        mesh=_sc_mesh(),
        compiler_params=_SC_PARAMS,
        scratch_types=[
            pltpu.VMEM((_ZC,), f32),
            pltpu.VMEM((_ZC,), f32),
            pltpu.VMEM((_ZC,), f32),
            pltpu.VMEM((_ZC,), f32),
        ],
    )
    x3 = final(p3, x1_full)
    return x3.reshape(_N_NODES, 1)


# speedup vs baseline: 432.6443x; 2.5365x over previous
import functools

import jax
import jax.numpy as jnp
from jax import lax
from jax.experimental import pallas as pl
from jax.experimental.pallas import tpu as pltpu
from jax.experimental.pallas import tpu_sc as plsc

_N_NODES = 100000
_N_EDGES = 6400000
_NUM_ET = 38
_HID = 128
_TBL = 640
_LANES = 128
_ROWS = _N_EDGES // _LANES
_CHUNK = 8
_NCHUNK = _ROWS // _CHUNK
_NW = 32
_ZC = 2000
_NZC = _N_NODES // _ZC


def _table_body(w1_ref, b1_ref, w2_ref, b2_ref, o_ref):
    row = lax.broadcasted_iota(jnp.int32, (_TBL, _HID), 0)
    col = lax.broadcasted_iota(jnp.int32, (_TBL, _HID), 1)
    et = row // 16
    ht = (row // 4) % 4
    tt = row % 4
    feat = ((col == et)
            | ((col >= _NUM_ET) & (col < _NUM_ET + 4) & (col - _NUM_ET == ht))
            | ((col >= _NUM_ET + 4) & (col < _NUM_ET + 8)
               & (col - (_NUM_ET + 4) == tt))).astype(jnp.float32)
    h = jnp.dot(feat, w1_ref[...], preferred_element_type=jnp.float32,
                precision="highest") + b1_ref[...]
    g = jax.nn.gelu(h)
    o = jnp.dot(g, w2_ref[...], preferred_element_type=jnp.float32,
                precision="highest") + b2_ref[...]
    o_ref[...] = jax.nn.sigmoid(o)


def _make_table(W1, b1, W2, b2):
    w1p = jnp.zeros((_HID, _HID), jnp.float32).at[: W1.shape[0]].set(W1)
    w2p = jnp.zeros((_HID, _HID), jnp.float32).at[:, :1].set(W2)
    b1r = b1.reshape(1, _HID)
    b2r = jnp.broadcast_to(b2.reshape(1, 1), (1, _HID))
    out = pl.pallas_call(
        _table_body,
        out_shape=jax.ShapeDtypeStruct((_TBL, _HID), jnp.float32),
    )(w1p, b1r, w2p, b2r)
    return out[:, 0]


def _worker_bounds(wid):
    base = _NCHUNK // _NW
    extra = _NCHUNK - base * _NW
    start = wid * base + jnp.minimum(wid, extra)
    cnt = base + jnp.where(wid < extra, 1, 0)
    return start, cnt


def _fill_zeros(zbuf):
    zeros16 = jnp.zeros((16,), jnp.float32)
    for i in range(_ZC // 16):
        zbuf[pl.ds(i * 16, 16)] = zeros16


def _zero_acc(acc, zbuf, sid):
    for j in range(4):
        k = sid + j * 16
        @pl.when(k < _NZC)
        def _():
            pltpu.sync_copy(zbuf, acc.at[pl.ds(k * _ZC, _ZC)])


def _write_partial(acc, zbuf, out_hbm, cid, sid):
    for j in range(4):
        k = sid + j * 16
        @pl.when(k < _NZC)
        def _():
            pltpu.sync_copy(acc.at[pl.ds(k * _ZC, _ZC)], zbuf)
            pltpu.sync_copy(zbuf, out_hbm.at[pl.ds(cid * _N_NODES + k * _ZC, _ZC)])


def _edge_pass(loads_hbm, loads_v, valsb, acc, wid, gather_vals, lsem, ssem):
    start, cnt = _worker_bounds(wid)

    def issue_loads(ci, p):
        r0 = (start + ci) * _CHUNK
        for h, v in zip(loads_hbm, loads_v[p]):
            pltpu.async_copy(h.at[pl.ds(r0, _CHUNK)], v, lsem)

    def wait_loads(p):
        for h, v in zip(loads_hbm, loads_v[p]):
            pltpu.make_async_copy(h.at[pl.ds(0, _CHUNK)], v, lsem).wait()

    def issue_scatters(p):
        dstb = loads_v[p][1]
        for j in range(_CHUNK):
            pltpu.async_copy(valsb[p].at[j], acc.at[dstb.at[j]], ssem,
                             add=True)

    def wait_scatters(p):
        dstb = loads_v[p][1]
        for j in range(_CHUNK):
            pltpu.make_async_copy(valsb[p].at[j], acc.at[dstb.at[j]],
                                  ssem).wait()

    issue_loads(0, 0)

    def pair_body(q, _):
        for p in (0, 1):
            ci = q * 2 + p
            @pl.when(ci < cnt)
            def _():
                wait_loads(p)
                @pl.when(ci > 0)
                def _():
                    wait_scatters(1 - p)
                @pl.when(ci + 1 < cnt)
                def _():
                    issue_loads(ci + 1, 1 - p)
                gather_vals(p)
                issue_scatters(p)
        return 0

    lax.fori_loop(0, (cnt + 1) // 2, pair_body, 0)

    @pl.when(lax.rem(cnt, 2) == 1)
    def _():
        wait_scatters(0)
    @pl.when(lax.rem(cnt, 2) == 0)
    def _():
        wait_scatters(1)


def _hop1_kernel(nt_hbm, src_hbm, dst_hbm, et_hbm, tbl_hbm, out_hbm,
                 nt_v, tbl_v,
                 srcb0, dstb0, etb0, srcb1, dstb1, etb1,
                 valsb0, valsb1, zbuf, acc, lsem, ssem):
    cid = lax.axis_index("c")
    sid = lax.axis_index("s")
    wid = sid * 2 + cid

    pltpu.sync_copy(nt_hbm, nt_v)
    pltpu.sync_copy(tbl_hbm, tbl_v)
    _fill_zeros(zbuf)
    _zero_acc(acc, zbuf, sid)
    plsc.subcore_barrier()

    loads_v = [[srcb0, dstb0, etb0], [srcb1, dstb1, etb1]]
    valsb = [valsb0, valsb1]

    def gather_vals(p):
        srcb, dstb, etb = loads_v[p]
        for j in range(_CHUNK):
            for b in range(_LANES // 16):
                s16 = srcb[j, pl.ds(b * 16, 16)]
                d16 = dstb[j, pl.ds(b * 16, 16)]
                e16 = etb[j, pl.ds(b * 16, 16)]
                ht = plsc.load_gather(nt_v, [s16])
                tt = plsc.load_gather(nt_v, [d16])
                combo = e16 * 16 + ht * 4 + tt
                valsb[p][j, pl.ds(b * 16, 16)] = plsc.load_gather(tbl_v, [combo])

    _edge_pass([src_hbm, dst_hbm, et_hbm], loads_v, valsb, acc, wid,
               gather_vals, lsem, ssem)
    plsc.subcore_barrier()
    _write_partial(acc, zbuf, out_hbm, cid, sid)


def _hop_kernel(nparts, emit_x, *refs):
    if nparts == 3:
        (src_hbm, dst_hbm, parts_hbm, extra_hbm, *rest) = refs
    else:
        (src_hbm, dst_hbm, parts_hbm, *rest) = refs
        extra_hbm = None
    if emit_x:
        (xout_hbm, out_hbm, *rest) = rest
    else:
        (out_hbm, *rest) = rest
        xout_hbm = None
    (x_v, srcb0, dstb0, srcb1, dstb1, valsb0, valsb1,
     t0a, t1a, t2a, t0b, t1b, t2b, zbuf, acc, lsem, ssem) = rest
    cid = lax.axis_index("c")
    sid = lax.axis_index("s")
    wid = sid * 2 + cid

    stage_bufs = [[t0a, t1a, t2a], [t0b, t1b, t2b]]

    def stage_issue(k, p):
        bufs = stage_bufs[p]
        pltpu.async_copy(parts_hbm.at[pl.ds(k * _ZC, _ZC)], bufs[0], lsem)
        pltpu.async_copy(parts_hbm.at[pl.ds(_N_NODES + k * _ZC, _ZC)],
                         bufs[1], lsem)
        if extra_hbm is not None:
            pltpu.async_copy(extra_hbm.at[pl.ds(k * _ZC, _ZC)], bufs[2], lsem)

    def stage_wait(p):
        bufs = stage_bufs[p]
        pltpu.make_async_copy(parts_hbm.at[pl.ds(0, _ZC)], bufs[0], lsem).wait()
        pltpu.make_async_copy(parts_hbm.at[pl.ds(0, _ZC)], bufs[1], lsem).wait()
        if extra_hbm is not None:
            pltpu.make_async_copy(extra_hbm.at[pl.ds(0, _ZC)], bufs[2],
                                  lsem).wait()

    stage_issue(0, 0)

    def stage_body(q, _):
        for p in (0, 1):
            k = q * 2 + p
            @pl.when(k < _NZC)
            def _():
                stage_wait(p)
                @pl.when(k + 1 < _NZC)
                def _():
                    stage_issue(k + 1, 1 - p)
                bufs = stage_bufs[p]
                for i in range(_ZC // 16):
                    v = bufs[0][pl.ds(i * 16, 16)] + bufs[1][pl.ds(i * 16, 16)]
                    if extra_hbm is not None:
                        v = v + bufs[2][pl.ds(i * 16, 16)]
                    x_v[pl.ds(k * _ZC + i * 16, 16)] = v
                if xout_hbm is not None:
                    @pl.when(lax.rem(k, _NW) == wid)
                    def _():
                        pltpu.sync_copy(x_v.at[pl.ds(k * _ZC, _ZC)],
                                        xout_hbm.at[pl.ds(k * _ZC, _ZC)])
        return 0

    lax.fori_loop(0, (_NZC + 1) // 2, stage_body, 0)

    _fill_zeros(zbuf)
    _zero_acc(acc, zbuf, sid)
    plsc.subcore_barrier()

    loads_v = [[srcb0, dstb0], [srcb1, dstb1]]
    valsb = [valsb0, valsb1]

    def gather_vals(p):
        srcb = loads_v[p][0]
        for j in range(_CHUNK):
            for b in range(_LANES // 16):
                s16 = srcb[j, pl.ds(b * 16, 16)]
                valsb[p][j, pl.ds(b * 16, 16)] = plsc.load_gather(x_v, [s16])

    _edge_pass([src_hbm, dst_hbm], loads_v, valsb, acc, wid, gather_vals,
               lsem, ssem)
    plsc.subcore_barrier()
    _write_partial(acc, zbuf, out_hbm, cid, sid)


def _final_kernel(p_hbm, x1_hbm, out_hbm, t0, t1, t2, sbuf):
    cid = lax.axis_index("c")
    sid = lax.axis_index("s")
    wid = sid * 2 + cid
    for j in range(2):
        k = wid + j * _NW
        @pl.when(k < _NZC)
        def _():
            pltpu.sync_copy(p_hbm.at[pl.ds(k * _ZC, _ZC)], t0)
            pltpu.sync_copy(p_hbm.at[pl.ds(_N_NODES + k * _ZC, _ZC)], t1)
            pltpu.sync_copy(x1_hbm.at[pl.ds(k * _ZC, _ZC)], t2)
            for i in range(_ZC // 16):
                sbuf[pl.ds(i * 16, 16)] = (t0[pl.ds(i * 16, 16)]
                                           + t1[pl.ds(i * 16, 16)]
                                           + t2[pl.ds(i * 16, 16)])
            pltpu.sync_copy(sbuf, out_hbm.at[pl.ds(k * _ZC, _ZC)])


def _sc_mesh():
    return plsc.VectorSubcoreMesh(core_axis_name="c", subcore_axis_name="s")


_SC_PARAMS = pltpu.CompilerParams(needs_layout_passes=False)


@jax.jit
def kernel(node_type, edge_index, edge_type, W1, b1, W2, b2):
    table = _make_table(W1, b1, W2, b2)

    src2d = edge_index[0].reshape(_ROWS, _LANES)
    dst2d = edge_index[1].reshape(_ROWS, _LANES)
    et2d = edge_type.reshape(_ROWS, _LANES)

    f32 = jnp.float32
    i32 = jnp.int32

    def ebuf(dt=i32):
        return pltpu.VMEM((_CHUNK, _LANES), dt)

    hop1 = pl.kernel(
        _hop1_kernel,
        out_type=jax.ShapeDtypeStruct((2 * _N_NODES,), f32),
        mesh=_sc_mesh(),
        compiler_params=_SC_PARAMS,
        scratch_types=[
            pltpu.VMEM((_N_NODES,), i32),
            pltpu.VMEM((_TBL,), f32),
            ebuf(), ebuf(), ebuf(),
            ebuf(), ebuf(), ebuf(),
            ebuf(f32), ebuf(f32),
            pltpu.VMEM((_ZC,), f32),
            pltpu.VMEM_SHARED((_N_NODES,), f32),
            pltpu.SemaphoreType.DMA,
            pltpu.SemaphoreType.DMA,
        ],
    )
    q = hop1(node_type, src2d, dst2d, et2d, table)

    hop_scratch = [
        pltpu.VMEM((_N_NODES,), f32),
        ebuf(), ebuf(),
        ebuf(), ebuf(),
        ebuf(f32), ebuf(f32),
        pltpu.VMEM((_ZC,), f32),
        pltpu.VMEM((_ZC,), f32),
        pltpu.VMEM((_ZC,), f32),
        pltpu.VMEM((_ZC,), f32),
        pltpu.VMEM((_ZC,), f32),
        pltpu.VMEM((_ZC,), f32),
        pltpu.VMEM((_ZC,), f32),
        pltpu.VMEM_SHARED((_N_NODES,), f32),
        pltpu.SemaphoreType.DMA,
        pltpu.SemaphoreType.DMA,
    ]

    hop2 = pl.kernel(
        functools.partial(_hop_kernel, 2, True),
        out_type=(jax.ShapeDtypeStruct((_N_NODES,), f32),
                  jax.ShapeDtypeStruct((2 * _N_NODES,), f32)),
        mesh=_sc_mesh(),
        compiler_params=_SC_PARAMS,
        scratch_types=list(hop_scratch),
    )
    x1_full, p2 = hop2(src2d, dst2d, q)

    hop3 = pl.kernel(
        functools.partial(_hop_kernel, 3, False),
        out_type=jax.ShapeDtypeStruct((2 * _N_NODES,), f32),
        mesh=_sc_mesh(),
        compiler_params=_SC_PARAMS,
        scratch_types=list(hop_scratch),
    )
    p3 = hop3(src2d, dst2d, p2, x1_full)

    final = pl.kernel(
        _final_kernel,
        out_type=jax.ShapeDtypeStruct((_N_NODES,), f32),
        mesh=_sc_mesh(),
        compiler_params=_SC_PARAMS,
        scratch_types=[
            pltpu.VMEM((_ZC,), f32),
            pltpu.VMEM((_ZC,), f32),
            pltpu.VMEM((_ZC,), f32),
            pltpu.VMEM((_ZC,), f32),
        ],
    )
    x3 = final(p3, x1_full)
    return x3.reshape(_N_NODES, 1)
